# Initial kernel scaffold; baseline (speedup 1.0000x reference)
import jax
import jax.numpy as jnp
from jax.experimental import pallas as pl


def kernel(x, raw_xyz, W_vw, b_vw, W_vote, b_vote, W_yaw, b_yaw, W_vel, b_vel, W_bin, b_bin, W_sres, b_sres, point2frameidx, frame2batchidx):
    raise NotImplementedError("write your pallas kernel here")


# speedup vs baseline: 1.4898x; 1.4898x over previous
import functools

import jax
import jax.numpy as jnp
from jax.experimental import pallas as pl
from jax.experimental.pallas import tpu as pltpu

N = 100000
FEAT = 256
NF = 320
NB = 32
NUM_SIZE_BINS = 12

R = 512


def _stage1_kernel(tb_ref, ts_ref, r0_ref, r1_ref,
                   x_ref, xyz_ref, wp_ref, bp_ref,
                   fmax_ref, sums_ref):
    t = pl.program_id(0)

    @pl.when(t == 0)
    def _init():
        fmax_ref[...] = jnp.full((NF, FEAT), -jnp.inf, jnp.float32)
        sums_ref[...] = jnp.zeros((NF, 8), jnp.float32)

    s = ts_ref[t]
    r0 = r0_ref[t]
    r1 = r1_ref[t]

    x = x_ref[...]
    rows = jax.lax.broadcasted_iota(jnp.int32, (R, 1), 0)
    inr = (rows >= r0) & (rows < r1)

    xm = jnp.where(inr, x, -jnp.inf)
    tmax = jnp.max(xm, axis=0, keepdims=True)
    old = fmax_ref[pl.ds(s, 1), :]
    fmax_ref[pl.ds(s, 1), :] = jnp.maximum(old, tmax)

    z = jax.lax.dot_general(x, wp_ref[...], (((1,), (1,)), ((), ())),
                            preferred_element_type=jnp.float32)
    z = z + bp_ref[...]
    w = jnp.clip(jax.nn.sigmoid(z[:, 0:1]), 1e-5)
    votes = (xyz_ref[...] + z[:, 1:4]) * w
    contrib = jnp.concatenate(
        [votes, w, jnp.zeros((R, 4), jnp.float32)], axis=1)
    contrib = jnp.where(inr, contrib, 0.0)
    csum = jnp.sum(contrib, axis=0, keepdims=True)
    sums_ref[pl.ds(s, 1), :] = sums_ref[pl.ds(s, 1), :] + csum


def _stage2_kernel(fmax_ref, sums_ref, f2b_ref, wf_ref, bf_ref, ws_ref, bs_ref,
                   cen_ref, vel_ref, yaw_ref, sres_ref, sbin_ref):
    fmax = fmax_ref[...]
    sums = sums_ref[...]

    mask = f2b_ref[...] == jax.lax.broadcasted_iota(jnp.int32, (NF, NB), 1)
    parts = []
    for j in range(NB):
        mj = jnp.max(jnp.where(mask[:, j:j + 1], fmax, -jnp.inf),
                     axis=0, keepdims=True)
        parts.append(mj)
    smax = jnp.concatenate(parts, axis=0)

    hf = jax.lax.dot_general(fmax, wf_ref[...], (((1,), (1,)), ((), ())),
                             preferred_element_type=jnp.float32)
    hf = hf + bf_ref[...]
    yaw_ref[...] = hf[:, 0:2]
    vel_ref[...] = hf[:, 2:5]

    hs = jax.lax.dot_general(smax, ws_ref[...], (((1,), (1,)), ((), ())),
                             preferred_element_type=jnp.float32)
    hs = hs + bs_ref[...]
    sres_ref[...] = hs[:, 0:NUM_SIZE_BINS * 3]
    binl = hs[:, NUM_SIZE_BINS * 3:NUM_SIZE_BINS * 4]
    m = jnp.max(binl, axis=1, keepdims=True)
    e = jnp.exp(binl - m)
    sbin_ref[...] = e / jnp.sum(e, axis=1, keepdims=True)

    cen_ref[...] = sums[:, 0:3] / sums[:, 3:4]


@functools.partial(jax.jit, static_argnames=())
def kernel(x, raw_xyz, W_vw, b_vw, W_vote, b_vote, W_yaw, b_yaw, W_vel, b_vel,
           W_bin, b_bin, W_sres, b_sres, point2frameidx, frame2batchidx):
    nblk = pl.cdiv(N, R)
    num_tiles = nblk + NF - 1

    ids = point2frameidx
    starts = jnp.searchsorted(ids, jnp.arange(NF + 1, dtype=jnp.int32)
                              ).astype(jnp.int32)
    bstart = jnp.arange(nblk, dtype=jnp.int32) * R
    blast = jnp.minimum(bstart + R, N) - 1
    blo = ids[bstart]
    bhi = ids[blast]
    cnt = bhi - blo + 1
    off = jnp.concatenate([jnp.zeros((1,), jnp.int32),
                           jnp.cumsum(cnt).astype(jnp.int32)])
    t_idx = jnp.arange(num_tiles, dtype=jnp.int32)
    tb = jnp.searchsorted(off, t_idx, side='right').astype(jnp.int32) - 1
    valid = t_idx < off[nblk]
    tb = jnp.clip(tb, 0, nblk - 1)
    ts = jnp.clip(blo[tb] + (t_idx - off[tb]), 0, NF - 1)
    r0 = jnp.clip(starts[ts] - tb * R, 0, R)
    r1 = jnp.clip(starts[ts + 1] - tb * R, 0, R)
    r0 = jnp.where(valid, r0, 0)
    r1 = jnp.where(valid, r1, 0)

    wp = jnp.zeros((8, FEAT), jnp.float32)
    wp = wp.at[0:1].set(W_vw).at[1:4].set(W_vote)
    bp = jnp.zeros((1, 8), jnp.float32)
    bp = bp.at[0, 0].set(b_vw[0]).at[0, 1:4].set(b_vote)

    grid_spec = pltpu.PrefetchScalarGridSpec(
        num_scalar_prefetch=4,
        grid=(num_tiles,),
        in_specs=[
            pl.BlockSpec((R, FEAT), lambda t, tb_, ts_, r0_, r1_: (tb_[t], 0)),
            pl.BlockSpec((R, 3), lambda t, tb_, ts_, r0_, r1_: (tb_[t], 0)),
            pl.BlockSpec((8, FEAT), lambda t, *_: (0, 0)),
            pl.BlockSpec((1, 8), lambda t, *_: (0, 0)),
        ],
        out_specs=[
            pl.BlockSpec((NF, FEAT), lambda t, *_: (0, 0)),
            pl.BlockSpec((NF, 8), lambda t, *_: (0, 0)),
        ],
    )
    fmax, sums = pl.pallas_call(
        _stage1_kernel,
        grid_spec=grid_spec,
        out_shape=[
            jax.ShapeDtypeStruct((NF, FEAT), jnp.float32),
            jax.ShapeDtypeStruct((NF, 8), jnp.float32),
        ],
    )(tb, ts, r0, r1, x, raw_xyz, wp, bp)

    wf = jnp.zeros((8, FEAT), jnp.float32)
    wf = wf.at[0:2].set(W_yaw).at[2:5].set(W_vel)
    bf = jnp.zeros((1, 8), jnp.float32)
    bf = bf.at[0, 0:2].set(b_yaw).at[0, 2:5].set(b_vel)
    ws = jnp.concatenate([W_sres, W_bin], axis=0)
    bs = jnp.concatenate([b_sres, b_bin])[None, :]
    f2b = frame2batchidx[:, None]

    centers, velocities, yaw, sres, sbin = pl.pallas_call(
        _stage2_kernel,
        in_specs=[
            pl.BlockSpec((NF, FEAT), lambda: (0, 0)),
            pl.BlockSpec((NF, 8), lambda: (0, 0)),
            pl.BlockSpec((NF, 1), lambda: (0, 0)),
            pl.BlockSpec((8, FEAT), lambda: (0, 0)),
            pl.BlockSpec((1, 8), lambda: (0, 0)),
            pl.BlockSpec((48, FEAT), lambda: (0, 0)),
            pl.BlockSpec((1, 48), lambda: (0, 0)),
        ],
        out_specs=[
            pl.BlockSpec((NF, 3), lambda: (0, 0)),
            pl.BlockSpec((NF, 3), lambda: (0, 0)),
            pl.BlockSpec((NF, 2), lambda: (0, 0)),
            pl.BlockSpec((NB, NUM_SIZE_BINS * 3), lambda: (0, 0)),
            pl.BlockSpec((NB, NUM_SIZE_BINS), lambda: (0, 0)),
        ],
        out_shape=[
            jax.ShapeDtypeStruct((NF, 3), jnp.float32),
            jax.ShapeDtypeStruct((NF, 3), jnp.float32),
            jax.ShapeDtypeStruct((NF, 2), jnp.float32),
            jax.ShapeDtypeStruct((NB, NUM_SIZE_BINS * 3), jnp.float32),
            jax.ShapeDtypeStruct((NB, NUM_SIZE_BINS), jnp.float32),
        ],
    )(fmax, sums, f2b, wf, bf, ws, bs)

    return (centers, velocities, yaw, sres, sbin)


# speedup vs baseline: 4.7999x; 3.2218x over previous
import functools

import jax
import jax.numpy as jnp
from jax.experimental import pallas as pl
from jax.experimental.pallas import tpu as pltpu

N = 100000
FEAT = 256
NF = 320
NB = 32
NUM_SIZE_BINS = 12

R = 4096
CH = 64
NCH = R // CH


def _stage1_kernel(starts_ref, blo_ref, bhi_ref,
                   x_ref, xyz_ref, wp_ref, bp_ref,
                   fmax_ref, sums_ref,
                   cmax_ref, csum_ref, contrib_ref):
    b = pl.program_id(0)

    @pl.when(b == 0)
    def _init():
        fmax_ref[...] = jnp.full((NF, FEAT), -jnp.inf, jnp.float32)
        sums_ref[...] = jnp.zeros((NF, 8), jnp.float32)

    x = x_ref[...]

    z = jax.lax.dot_general(x, wp_ref[...], (((1,), (1,)), ((), ())),
                            preferred_element_type=jnp.float32)
    z = z + bp_ref[...]
    w = jnp.clip(jax.nn.sigmoid(z[:, 0:1]), 1e-5)
    votes = (xyz_ref[...] + z[:, 1:4]) * w
    contrib = jnp.concatenate(
        [votes, w, jnp.zeros((R, 4), jnp.float32)], axis=1)
    contrib_ref[...] = contrib
    cmax_ref[...] = jnp.max(x.reshape(NCH, CH, FEAT), axis=1)
    csum_ref[...] = jnp.sum(contrib.reshape(NCH, CH, 8), axis=1)

    base = b * R
    ci = jax.lax.broadcasted_iota(jnp.int32, (NCH, 1), 0)
    rows = jax.lax.broadcasted_iota(jnp.int32, (CH, 1), 0)

    def seg_body(s, _):
        r0 = jnp.maximum(starts_ref[s] - base, 0)
        r1 = jnp.minimum(starts_ref[s + 1] - base, R)
        ch0 = jax.lax.div(r0, CH)
        chl = jax.lax.div(jnp.maximum(r1, 1) - 1, CH)

        inner = (ci > ch0) & (ci < chl)
        m_int = jnp.max(jnp.where(inner, cmax_ref[...], -jnp.inf),
                        axis=0, keepdims=True)
        s_int = jnp.sum(jnp.where(inner, csum_ref[...], 0.0),
                        axis=0, keepdims=True)

        rh = rows + ch0 * CH
        mh = (rh >= r0) & (rh < r1)
        xh = x_ref[pl.ds(ch0 * CH, CH), :]
        m_h = jnp.max(jnp.where(mh, xh, -jnp.inf), axis=0, keepdims=True)
        s_h = jnp.sum(jnp.where(mh, contrib_ref[pl.ds(ch0 * CH, CH), :], 0.0),
                      axis=0, keepdims=True)

        rt = rows + chl * CH
        mt = (rt >= r0) & (rt < r1) & (chl > ch0)
        xt = x_ref[pl.ds(chl * CH, CH), :]
        m_t = jnp.max(jnp.where(mt, xt, -jnp.inf), axis=0, keepdims=True)
        s_t = jnp.sum(jnp.where(mt, contrib_ref[pl.ds(chl * CH, CH), :], 0.0),
                      axis=0, keepdims=True)

        old_m = fmax_ref[pl.ds(s, 1), :]
        fmax_ref[pl.ds(s, 1), :] = jnp.maximum(
            jnp.maximum(old_m, m_int), jnp.maximum(m_h, m_t))
        sums_ref[pl.ds(s, 1), :] = (sums_ref[pl.ds(s, 1), :]
                                    + s_int + s_h + s_t)
        return 0

    jax.lax.fori_loop(blo_ref[b], bhi_ref[b] + 1, seg_body, 0)


def _stage2_kernel(fmax_ref, sums_ref, f2b_ref, wf_ref, bf_ref, ws_ref, bs_ref,
                   cen_ref, vel_ref, yaw_ref, sres_ref, sbin_ref):
    fmax = fmax_ref[...]
    sums = sums_ref[...]

    mask = f2b_ref[...] == jax.lax.broadcasted_iota(jnp.int32, (NF, NB), 1)
    parts = []
    for j in range(NB):
        mj = jnp.max(jnp.where(mask[:, j:j + 1], fmax, -jnp.inf),
                     axis=0, keepdims=True)
        parts.append(mj)
    smax = jnp.concatenate(parts, axis=0)

    hf = jax.lax.dot_general(fmax, wf_ref[...], (((1,), (1,)), ((), ())),
                             preferred_element_type=jnp.float32)
    hf = hf + bf_ref[...]
    yaw_ref[...] = hf[:, 0:2]
    vel_ref[...] = hf[:, 2:5]

    hs = jax.lax.dot_general(smax, ws_ref[...], (((1,), (1,)), ((), ())),
                             preferred_element_type=jnp.float32)
    hs = hs + bs_ref[...]
    sres_ref[...] = hs[:, 0:NUM_SIZE_BINS * 3]
    binl = hs[:, NUM_SIZE_BINS * 3:NUM_SIZE_BINS * 4]
    m = jnp.max(binl, axis=1, keepdims=True)
    e = jnp.exp(binl - m)
    sbin_ref[...] = e / jnp.sum(e, axis=1, keepdims=True)

    cen_ref[...] = sums[:, 0:3] / sums[:, 3:4]


@jax.jit
def kernel(x, raw_xyz, W_vw, b_vw, W_vote, b_vote, W_yaw, b_yaw, W_vel, b_vel,
           W_bin, b_bin, W_sres, b_sres, point2frameidx, frame2batchidx):
    nblk = pl.cdiv(N, R)

    ids = point2frameidx
    starts = jnp.searchsorted(ids, jnp.arange(NF + 1, dtype=jnp.int32)
                              ).astype(jnp.int32)
    bstart = jnp.arange(nblk, dtype=jnp.int32) * R
    blast = jnp.minimum(bstart + R, N) - 1
    blo = ids[bstart]
    bhi = ids[blast]

    wp = jnp.zeros((8, FEAT), jnp.float32)
    wp = wp.at[0:1].set(W_vw).at[1:4].set(W_vote)
    bp = jnp.zeros((1, 8), jnp.float32)
    bp = bp.at[0, 0].set(b_vw[0]).at[0, 1:4].set(b_vote)

    grid_spec = pltpu.PrefetchScalarGridSpec(
        num_scalar_prefetch=3,
        grid=(nblk,),
        in_specs=[
            pl.BlockSpec((R, FEAT), lambda b, *_: (b, 0)),
            pl.BlockSpec((R, 3), lambda b, *_: (b, 0)),
            pl.BlockSpec((8, FEAT), lambda b, *_: (0, 0)),
            pl.BlockSpec((1, 8), lambda b, *_: (0, 0)),
        ],
        out_specs=[
            pl.BlockSpec((NF, FEAT), lambda b, *_: (0, 0)),
            pl.BlockSpec((NF, 8), lambda b, *_: (0, 0)),
        ],
        scratch_shapes=[
            pltpu.VMEM((NCH, FEAT), jnp.float32),
            pltpu.VMEM((NCH, 8), jnp.float32),
            pltpu.VMEM((R, 8), jnp.float32),
        ],
    )
    fmax, sums = pl.pallas_call(
        _stage1_kernel,
        grid_spec=grid_spec,
        out_shape=[
            jax.ShapeDtypeStruct((NF, FEAT), jnp.float32),
            jax.ShapeDtypeStruct((NF, 8), jnp.float32),
        ],
    )(starts, blo, bhi, x, raw_xyz, wp, bp)

    wf = jnp.zeros((8, FEAT), jnp.float32)
    wf = wf.at[0:2].set(W_yaw).at[2:5].set(W_vel)
    bf = jnp.zeros((1, 8), jnp.float32)
    bf = bf.at[0, 0:2].set(b_yaw).at[0, 2:5].set(b_vel)
    ws = jnp.concatenate([W_sres, W_bin], axis=0)
    bs = jnp.concatenate([b_sres, b_bin])[None, :]
    f2b = frame2batchidx[:, None]

    centers, velocities, yaw, sres, sbin = pl.pallas_call(
        _stage2_kernel,
        in_specs=[
            pl.BlockSpec((NF, FEAT), lambda: (0, 0)),
            pl.BlockSpec((NF, 8), lambda: (0, 0)),
            pl.BlockSpec((NF, 1), lambda: (0, 0)),
            pl.BlockSpec((8, FEAT), lambda: (0, 0)),
            pl.BlockSpec((1, 8), lambda: (0, 0)),
            pl.BlockSpec((48, FEAT), lambda: (0, 0)),
            pl.BlockSpec((1, 48), lambda: (0, 0)),
        ],
        out_specs=[
            pl.BlockSpec((NF, 3), lambda: (0, 0)),
            pl.BlockSpec((NF, 3), lambda: (0, 0)),
            pl.BlockSpec((NF, 2), lambda: (0, 0)),
            pl.BlockSpec((NB, NUM_SIZE_BINS * 3), lambda: (0, 0)),
            pl.BlockSpec((NB, NUM_SIZE_BINS), lambda: (0, 0)),
        ],
        out_shape=[
            jax.ShapeDtypeStruct((NF, 3), jnp.float32),
            jax.ShapeDtypeStruct((NF, 3), jnp.float32),
            jax.ShapeDtypeStruct((NF, 2), jnp.float32),
            jax.ShapeDtypeStruct((NB, NUM_SIZE_BINS * 3), jnp.float32),
            jax.ShapeDtypeStruct((NB, NUM_SIZE_BINS), jnp.float32),
        ],
    )(fmax, sums, f2b, wf, bf, ws, bs)

    return (centers, velocities, yaw, sres, sbin)


# speedup vs baseline: 5.7718x; 1.2025x over previous
import functools

import jax
import jax.numpy as jnp
from jax.experimental import pallas as pl
from jax.experimental.pallas import tpu as pltpu

N = 100000
FEAT = 256
NF = 320
NB = 32
NUM_SIZE_BINS = 12

R = 4096
CH = 64
NCH = R // CH


def _stage1_kernel(starts_ref, blo_ref, bhi_ref,
                   x_ref, xyzt_ref, wp_ref, bpt_ref,
                   fmax_ref, sums_ref,
                   cmax_ref, csum_ref, contrib_ref):
    b = pl.program_id(0)

    @pl.when(b == 0)
    def _init():
        fmax_ref[...] = jnp.full((NF, FEAT), -jnp.inf, jnp.float32)
        sums_ref[...] = jnp.zeros((NF, 4), jnp.float32)

    x = x_ref[...]

    zt = jax.lax.dot_general(wp_ref[...], x, (((1,), (1,)), ((), ())),
                             preferred_element_type=jnp.float32)
    zt = zt + bpt_ref[...]
    wt = jnp.clip(jax.nn.sigmoid(zt[0:1, :]), 1e-5)
    votest = (xyzt_ref[0:3, :] + zt[1:4, :]) * wt
    contribt = jnp.concatenate([votest, wt], axis=0)
    contrib = contribt.T
    contrib_ref[...] = contrib
    cmax_ref[...] = jnp.max(x.reshape(NCH, CH, FEAT), axis=1)
    csum_ref[...] = jnp.sum(contrib.reshape(NCH, CH, 4), axis=1)

    base = b * R
    ci = jax.lax.broadcasted_iota(jnp.int32, (NCH, 1), 0)
    rows = jax.lax.broadcasted_iota(jnp.int32, (CH, 1), 0)

    def seg_body(s, _):
        r0 = jnp.maximum(starts_ref[s] - base, 0)
        r1 = jnp.minimum(starts_ref[s + 1] - base, R)
        ch0 = jax.lax.div(r0, CH)
        chl = jax.lax.div(jnp.maximum(r1, 1) - 1, CH)

        inner = (ci > ch0) & (ci < chl)
        m_int = jnp.max(jnp.where(inner, cmax_ref[...], -jnp.inf),
                        axis=0, keepdims=True)
        s_int = jnp.sum(jnp.where(inner, csum_ref[...], 0.0),
                        axis=0, keepdims=True)

        rh = rows + ch0 * CH
        mh = (rh >= r0) & (rh < r1)
        xh = x_ref[pl.ds(ch0 * CH, CH), :]
        m_h = jnp.max(jnp.where(mh, xh, -jnp.inf), axis=0, keepdims=True)
        s_h = jnp.sum(jnp.where(mh, contrib_ref[pl.ds(ch0 * CH, CH), :], 0.0),
                      axis=0, keepdims=True)

        rt = rows + chl * CH
        mt = (rt >= r0) & (rt < r1) & (chl > ch0)
        xt = x_ref[pl.ds(chl * CH, CH), :]
        m_t = jnp.max(jnp.where(mt, xt, -jnp.inf), axis=0, keepdims=True)
        s_t = jnp.sum(jnp.where(mt, contrib_ref[pl.ds(chl * CH, CH), :], 0.0),
                      axis=0, keepdims=True)

        old_m = fmax_ref[pl.ds(s, 1), :]
        fmax_ref[pl.ds(s, 1), :] = jnp.maximum(
            jnp.maximum(old_m, m_int), jnp.maximum(m_h, m_t))
        sums_ref[pl.ds(s, 1), :] = (sums_ref[pl.ds(s, 1), :]
                                    + s_int + s_h + s_t)
        return 0

    jax.lax.fori_loop(blo_ref[b], bhi_ref[b] + 1, seg_body, 0)


def _stage2_kernel(fmax_ref, sums_ref, f2b_ref, wf_ref, bf_ref, ws_ref, bs_ref,
                   cen_ref, vel_ref, yaw_ref, sres_ref, sbin_ref):
    fmax = fmax_ref[...]
    sums = sums_ref[...]

    mask = f2b_ref[...] == jax.lax.broadcasted_iota(jnp.int32, (NF, NB), 1)
    parts = []
    for j in range(NB):
        mj = jnp.max(jnp.where(mask[:, j:j + 1], fmax, -jnp.inf),
                     axis=0, keepdims=True)
        parts.append(mj)
    smax = jnp.concatenate(parts, axis=0)

    hf = jax.lax.dot_general(fmax, wf_ref[...], (((1,), (1,)), ((), ())),
                             preferred_element_type=jnp.float32)
    hf = hf + bf_ref[...]
    yaw_ref[...] = hf[:, 0:2]
    vel_ref[...] = hf[:, 2:5]

    hs = jax.lax.dot_general(smax, ws_ref[...], (((1,), (1,)), ((), ())),
                             preferred_element_type=jnp.float32)
    hs = hs + bs_ref[...]
    sres_ref[...] = hs[:, 0:NUM_SIZE_BINS * 3]
    binl = hs[:, NUM_SIZE_BINS * 3:NUM_SIZE_BINS * 4]
    m = jnp.max(binl, axis=1, keepdims=True)
    e = jnp.exp(binl - m)
    sbin_ref[...] = e / jnp.sum(e, axis=1, keepdims=True)

    cen_ref[...] = sums[:, 0:3] / sums[:, 3:4]


@jax.jit
def kernel(x, raw_xyz, W_vw, b_vw, W_vote, b_vote, W_yaw, b_yaw, W_vel, b_vel,
           W_bin, b_bin, W_sres, b_sres, point2frameidx, frame2batchidx):
    nblk = pl.cdiv(N, R)

    ids = point2frameidx
    starts = jnp.searchsorted(ids, jnp.arange(NF + 1, dtype=jnp.int32)
                              ).astype(jnp.int32)
    bstart = jnp.arange(nblk, dtype=jnp.int32) * R
    blast = jnp.minimum(bstart + R, N) - 1
    blo = ids[bstart]
    bhi = ids[blast]

    wp = jnp.zeros((8, FEAT), jnp.float32)
    wp = wp.at[0:1].set(W_vw).at[1:4].set(W_vote)
    bpt = jnp.zeros((8, 1), jnp.float32)
    bpt = bpt.at[0, 0].set(b_vw[0]).at[1:4, 0].set(b_vote)
    xyzt = jnp.zeros((8, nblk * R), jnp.float32).at[0:3, :N].set(raw_xyz.T)

    grid_spec = pltpu.PrefetchScalarGridSpec(
        num_scalar_prefetch=3,
        grid=(nblk,),
        in_specs=[
            pl.BlockSpec((R, FEAT), lambda b, *_: (b, 0)),
            pl.BlockSpec((8, R), lambda b, *_: (0, b)),
            pl.BlockSpec((8, FEAT), lambda b, *_: (0, 0)),
            pl.BlockSpec((8, 1), lambda b, *_: (0, 0)),
        ],
        out_specs=[
            pl.BlockSpec((NF, FEAT), lambda b, *_: (0, 0)),
            pl.BlockSpec((NF, 4), lambda b, *_: (0, 0)),
        ],
        scratch_shapes=[
            pltpu.VMEM((NCH, FEAT), jnp.float32),
            pltpu.VMEM((NCH, 4), jnp.float32),
            pltpu.VMEM((R, 4), jnp.float32),
        ],
    )
    fmax, sums = pl.pallas_call(
        _stage1_kernel,
        grid_spec=grid_spec,
        out_shape=[
            jax.ShapeDtypeStruct((NF, FEAT), jnp.float32),
            jax.ShapeDtypeStruct((NF, 4), jnp.float32),
        ],
    )(starts, blo, bhi, x, xyzt, wp, bpt)

    wf = jnp.zeros((8, FEAT), jnp.float32)
    wf = wf.at[0:2].set(W_yaw).at[2:5].set(W_vel)
    bf = jnp.zeros((1, 8), jnp.float32)
    bf = bf.at[0, 0:2].set(b_yaw).at[0, 2:5].set(b_vel)
    ws = jnp.concatenate([W_sres, W_bin], axis=0)
    bs = jnp.concatenate([b_sres, b_bin])[None, :]
    f2b = frame2batchidx[:, None]

    centers, velocities, yaw, sres, sbin = pl.pallas_call(
        _stage2_kernel,
        in_specs=[
            pl.BlockSpec((NF, FEAT), lambda: (0, 0)),
            pl.BlockSpec((NF, 4), lambda: (0, 0)),
            pl.BlockSpec((NF, 1), lambda: (0, 0)),
            pl.BlockSpec((8, FEAT), lambda: (0, 0)),
            pl.BlockSpec((1, 8), lambda: (0, 0)),
            pl.BlockSpec((48, FEAT), lambda: (0, 0)),
            pl.BlockSpec((1, 48), lambda: (0, 0)),
        ],
        out_specs=[
            pl.BlockSpec((NF, 3), lambda: (0, 0)),
            pl.BlockSpec((NF, 3), lambda: (0, 0)),
            pl.BlockSpec((NF, 2), lambda: (0, 0)),
            pl.BlockSpec((NB, NUM_SIZE_BINS * 3), lambda: (0, 0)),
            pl.BlockSpec((NB, NUM_SIZE_BINS), lambda: (0, 0)),
        ],
        out_shape=[
            jax.ShapeDtypeStruct((NF, 3), jnp.float32),
            jax.ShapeDtypeStruct((NF, 3), jnp.float32),
            jax.ShapeDtypeStruct((NF, 2), jnp.float32),
            jax.ShapeDtypeStruct((NB, NUM_SIZE_BINS * 3), jnp.float32),
            jax.ShapeDtypeStruct((NB, NUM_SIZE_BINS), jnp.float32),
        ],
    )(fmax, sums, f2b, wf, bf, ws, bs)

    return (centers, velocities, yaw, sres, sbin)


# speedup vs baseline: 5.7727x; 1.0002x over previous
import functools

import jax
import jax.numpy as jnp
from jax.experimental import pallas as pl
from jax.experimental.pallas import tpu as pltpu

N = 100000
FEAT = 256
NF = 320
NB = 32
NUM_SIZE_BINS = 12

R = 4096
CH = 64
NCH = R // CH


def _stage1_kernel(starts_ref, blo_ref, bhi_ref,
                   x_ref, xyzt_ref, wp_ref, bpt_ref,
                   fmax_out_ref, sums_out_ref,
                   cmax_ref, csum_ref, contrib_ref, fmax_ref, sums_ref):
    b = pl.program_id(0)
    nblk = pl.num_programs(0)

    @pl.when(b == 0)
    def _init():
        fmax_ref[...] = jnp.full((NF, FEAT), -jnp.inf, jnp.float32)
        sums_ref[...] = jnp.zeros((NF, 4), jnp.float32)

    x = x_ref[...]

    zt = jax.lax.dot_general(wp_ref[...], x, (((1,), (1,)), ((), ())),
                             preferred_element_type=jnp.float32)
    zt = zt + bpt_ref[...]
    wt = jnp.clip(jax.nn.sigmoid(zt[0:1, :]), 1e-5)
    votest = (xyzt_ref[0:3, :] + zt[1:4, :]) * wt
    contribt = jnp.concatenate([votest, wt], axis=0)
    contrib = contribt.T
    contrib_ref[...] = contrib
    cmax_ref[...] = jnp.max(x.reshape(NCH, CH, FEAT), axis=1)
    csum_ref[...] = jnp.sum(contrib.reshape(NCH, CH, 4), axis=1)

    base = b * R
    ci = jax.lax.broadcasted_iota(jnp.int32, (NCH, 1), 0)
    rows = jax.lax.broadcasted_iota(jnp.int32, (CH, 1), 0)

    def seg_body(s, _):
        r0 = jnp.maximum(starts_ref[s] - base, 0)
        r1 = jnp.minimum(starts_ref[s + 1] - base, R)
        ch0 = jax.lax.div(r0, CH)
        chl = jax.lax.div(jnp.maximum(r1, 1) - 1, CH)

        inner = (ci > ch0) & (ci < chl)
        m_int = jnp.max(jnp.where(inner, cmax_ref[...], -jnp.inf),
                        axis=0, keepdims=True)
        s_int = jnp.sum(jnp.where(inner, csum_ref[...], 0.0),
                        axis=0, keepdims=True)

        rh = rows + ch0 * CH
        mh = (rh >= r0) & (rh < r1)
        xh = x_ref[pl.ds(ch0 * CH, CH), :]
        m_h = jnp.max(jnp.where(mh, xh, -jnp.inf), axis=0, keepdims=True)
        s_h = jnp.sum(jnp.where(mh, contrib_ref[pl.ds(ch0 * CH, CH), :], 0.0),
                      axis=0, keepdims=True)

        rt = rows + chl * CH
        mt = (rt >= r0) & (rt < r1) & (chl > ch0)
        xt = x_ref[pl.ds(chl * CH, CH), :]
        m_t = jnp.max(jnp.where(mt, xt, -jnp.inf), axis=0, keepdims=True)
        s_t = jnp.sum(jnp.where(mt, contrib_ref[pl.ds(chl * CH, CH), :], 0.0),
                      axis=0, keepdims=True)

        old_m = fmax_ref[pl.ds(s, 1), :]
        fmax_ref[pl.ds(s, 1), :] = jnp.maximum(
            jnp.maximum(old_m, m_int), jnp.maximum(m_h, m_t))
        sums_ref[pl.ds(s, 1), :] = (sums_ref[pl.ds(s, 1), :]
                                    + s_int + s_h + s_t)
        return 0

    jax.lax.fori_loop(blo_ref[b], bhi_ref[b] + 1, seg_body, 0)

    @pl.when(b == nblk - 1)
    def _flush():
        fmax_out_ref[...] = fmax_ref[...]
        sums_out_ref[...] = sums_ref[...]


def _stage2_kernel(fmax_ref, sums_ref, f2b_ref, wf_ref, bf_ref, ws_ref, bs_ref,
                   cen_ref, vel_ref, yaw_ref, sres_ref, sbin_ref):
    fmax = fmax_ref[...]
    sums = sums_ref[...]

    mask = f2b_ref[...] == jax.lax.broadcasted_iota(jnp.int32, (NF, NB), 1)
    parts = []
    for j in range(NB):
        mj = jnp.max(jnp.where(mask[:, j:j + 1], fmax, -jnp.inf),
                     axis=0, keepdims=True)
        parts.append(mj)
    smax = jnp.concatenate(parts, axis=0)

    hf = jax.lax.dot_general(fmax, wf_ref[...], (((1,), (1,)), ((), ())),
                             preferred_element_type=jnp.float32)
    hf = hf + bf_ref[...]
    yaw_ref[...] = hf[:, 0:2]
    vel_ref[...] = hf[:, 2:5]

    hs = jax.lax.dot_general(smax, ws_ref[...], (((1,), (1,)), ((), ())),
                             preferred_element_type=jnp.float32)
    hs = hs + bs_ref[...]
    sres_ref[...] = hs[:, 0:NUM_SIZE_BINS * 3]
    binl = hs[:, NUM_SIZE_BINS * 3:NUM_SIZE_BINS * 4]
    m = jnp.max(binl, axis=1, keepdims=True)
    e = jnp.exp(binl - m)
    sbin_ref[...] = e / jnp.sum(e, axis=1, keepdims=True)

    cen_ref[...] = sums[:, 0:3] / sums[:, 3:4]


@jax.jit
def kernel(x, raw_xyz, W_vw, b_vw, W_vote, b_vote, W_yaw, b_yaw, W_vel, b_vel,
           W_bin, b_bin, W_sres, b_sres, point2frameidx, frame2batchidx):
    nblk = pl.cdiv(N, R)

    ids = point2frameidx
    starts = jnp.searchsorted(ids, jnp.arange(NF + 1, dtype=jnp.int32)
                              ).astype(jnp.int32)
    bstart = jnp.arange(nblk, dtype=jnp.int32) * R
    blast = jnp.minimum(bstart + R, N) - 1
    blo = ids[bstart]
    bhi = ids[blast]

    wp = jnp.zeros((8, FEAT), jnp.float32)
    wp = wp.at[0:1].set(W_vw).at[1:4].set(W_vote)
    bpt = jnp.zeros((8, 1), jnp.float32)
    bpt = bpt.at[0, 0].set(b_vw[0]).at[1:4, 0].set(b_vote)
    xyzt = jnp.zeros((8, nblk * R), jnp.float32).at[0:3, :N].set(raw_xyz.T)

    grid_spec = pltpu.PrefetchScalarGridSpec(
        num_scalar_prefetch=3,
        grid=(nblk,),
        in_specs=[
            pl.BlockSpec((R, FEAT), lambda b, *_: (b, 0)),
            pl.BlockSpec((8, R), lambda b, *_: (0, b)),
            pl.BlockSpec((8, FEAT), lambda b, *_: (0, 0)),
            pl.BlockSpec((8, 1), lambda b, *_: (0, 0)),
        ],
        out_specs=[
            pl.BlockSpec((NF, FEAT), lambda b, *_: (0, 0)),
            pl.BlockSpec((NF, 4), lambda b, *_: (0, 0)),
        ],
        scratch_shapes=[
            pltpu.VMEM((NCH, FEAT), jnp.float32),
            pltpu.VMEM((NCH, 4), jnp.float32),
            pltpu.VMEM((R, 4), jnp.float32),
            pltpu.VMEM((NF, FEAT), jnp.float32),
            pltpu.VMEM((NF, 4), jnp.float32),
        ],
    )
    fmax, sums = pl.pallas_call(
        _stage1_kernel,
        grid_spec=grid_spec,
        out_shape=[
            jax.ShapeDtypeStruct((NF, FEAT), jnp.float32),
            jax.ShapeDtypeStruct((NF, 4), jnp.float32),
        ],
    )(starts, blo, bhi, x, xyzt, wp, bpt)

    wf = jnp.zeros((8, FEAT), jnp.float32)
    wf = wf.at[0:2].set(W_yaw).at[2:5].set(W_vel)
    bf = jnp.zeros((1, 8), jnp.float32)
    bf = bf.at[0, 0:2].set(b_yaw).at[0, 2:5].set(b_vel)
    ws = jnp.concatenate([W_sres, W_bin], axis=0)
    bs = jnp.concatenate([b_sres, b_bin])[None, :]
    f2b = frame2batchidx[:, None]

    centers, velocities, yaw, sres, sbin = pl.pallas_call(
        _stage2_kernel,
        in_specs=[
            pl.BlockSpec((NF, FEAT), lambda: (0, 0)),
            pl.BlockSpec((NF, 4), lambda: (0, 0)),
            pl.BlockSpec((NF, 1), lambda: (0, 0)),
            pl.BlockSpec((8, FEAT), lambda: (0, 0)),
            pl.BlockSpec((1, 8), lambda: (0, 0)),
            pl.BlockSpec((48, FEAT), lambda: (0, 0)),
            pl.BlockSpec((1, 48), lambda: (0, 0)),
        ],
        out_specs=[
            pl.BlockSpec((NF, 3), lambda: (0, 0)),
            pl.BlockSpec((NF, 3), lambda: (0, 0)),
            pl.BlockSpec((NF, 2), lambda: (0, 0)),
            pl.BlockSpec((NB, NUM_SIZE_BINS * 3), lambda: (0, 0)),
            pl.BlockSpec((NB, NUM_SIZE_BINS), lambda: (0, 0)),
        ],
        out_shape=[
            jax.ShapeDtypeStruct((NF, 3), jnp.float32),
            jax.ShapeDtypeStruct((NF, 3), jnp.float32),
            jax.ShapeDtypeStruct((NF, 2), jnp.float32),
            jax.ShapeDtypeStruct((NB, NUM_SIZE_BINS * 3), jnp.float32),
            jax.ShapeDtypeStruct((NB, NUM_SIZE_BINS), jnp.float32),
        ],
    )(fmax, sums, f2b, wf, bf, ws, bs)

    return (centers, velocities, yaw, sres, sbin)


# speedup vs baseline: 5.9120x; 1.0241x over previous
import functools

import jax
import jax.numpy as jnp
from jax.experimental import pallas as pl
from jax.experimental.pallas import tpu as pltpu

N = 100000
FEAT = 256
NF = 320
NB = 32
NUM_SIZE_BINS = 12

R = 8192
CH = 64
NCH = R // CH


def _stage1_kernel(starts_ref, blo_ref, bhi_ref,
                   x_ref, xyzt_ref, wp_ref, bpt_ref,
                   fmax_out_ref, sums_out_ref,
                   cmax_ref, csum_ref, contrib_ref, fmax_ref, sums_ref):
    b = pl.program_id(0)
    nblk = pl.num_programs(0)

    @pl.when(b == 0)
    def _init():
        fmax_ref[...] = jnp.full((NF, FEAT), -jnp.inf, jnp.float32)
        sums_ref[...] = jnp.zeros((NF, 4), jnp.float32)

    x = x_ref[...]

    zt = jax.lax.dot_general(wp_ref[...], x, (((1,), (1,)), ((), ())),
                             preferred_element_type=jnp.float32)
    zt = zt + bpt_ref[...]
    wt = jnp.clip(jax.nn.sigmoid(zt[0:1, :]), 1e-5)
    votest = (xyzt_ref[0:3, :] + zt[1:4, :]) * wt
    contribt = jnp.concatenate([votest, wt], axis=0)
    contrib = contribt.T
    contrib_ref[...] = contrib
    cmax_ref[...] = jnp.max(x.reshape(NCH, CH, FEAT), axis=1)
    csum_ref[...] = jnp.sum(contrib.reshape(NCH, CH, 4), axis=1)

    base = b * R
    ci = jax.lax.broadcasted_iota(jnp.int32, (NCH, 1), 0)
    rows = jax.lax.broadcasted_iota(jnp.int32, (CH, 1), 0)

    def seg_body(s, _):
        r0 = jnp.maximum(starts_ref[s] - base, 0)
        r1 = jnp.minimum(starts_ref[s + 1] - base, R)
        ch0 = jax.lax.div(r0, CH)
        chl = jax.lax.div(jnp.maximum(r1, 1) - 1, CH)

        inner = (ci > ch0) & (ci < chl)
        m_int = jnp.max(jnp.where(inner, cmax_ref[...], -jnp.inf),
                        axis=0, keepdims=True)
        s_int = jnp.sum(jnp.where(inner, csum_ref[...], 0.0),
                        axis=0, keepdims=True)

        rh = rows + ch0 * CH
        mh = (rh >= r0) & (rh < r1)
        xh = x_ref[pl.ds(ch0 * CH, CH), :]
        m_h = jnp.max(jnp.where(mh, xh, -jnp.inf), axis=0, keepdims=True)
        s_h = jnp.sum(jnp.where(mh, contrib_ref[pl.ds(ch0 * CH, CH), :], 0.0),
                      axis=0, keepdims=True)

        rt = rows + chl * CH
        mt = (rt >= r0) & (rt < r1) & (chl > ch0)
        xt = x_ref[pl.ds(chl * CH, CH), :]
        m_t = jnp.max(jnp.where(mt, xt, -jnp.inf), axis=0, keepdims=True)
        s_t = jnp.sum(jnp.where(mt, contrib_ref[pl.ds(chl * CH, CH), :], 0.0),
                      axis=0, keepdims=True)

        old_m = fmax_ref[pl.ds(s, 1), :]
        fmax_ref[pl.ds(s, 1), :] = jnp.maximum(
            jnp.maximum(old_m, m_int), jnp.maximum(m_h, m_t))
        sums_ref[pl.ds(s, 1), :] = (sums_ref[pl.ds(s, 1), :]
                                    + s_int + s_h + s_t)
        return 0

    jax.lax.fori_loop(blo_ref[b], bhi_ref[b] + 1, seg_body, 0)

    @pl.when(b == nblk - 1)
    def _flush():
        fmax_out_ref[...] = fmax_ref[...]
        sums_out_ref[...] = sums_ref[...]


def _stage2_kernel(fmax_ref, sums_ref, f2b_ref, wf_ref, bf_ref, ws_ref, bs_ref,
                   cen_ref, vel_ref, yaw_ref, sres_ref, sbin_ref):
    fmax = fmax_ref[...]
    sums = sums_ref[...]

    mask = f2b_ref[...] == jax.lax.broadcasted_iota(jnp.int32, (NF, NB), 1)
    parts = []
    for j in range(NB):
        mj = jnp.max(jnp.where(mask[:, j:j + 1], fmax, -jnp.inf),
                     axis=0, keepdims=True)
        parts.append(mj)
    smax = jnp.concatenate(parts, axis=0)

    hf = jax.lax.dot_general(fmax, wf_ref[...], (((1,), (1,)), ((), ())),
                             preferred_element_type=jnp.float32)
    hf = hf + bf_ref[...]
    yaw_ref[...] = hf[:, 0:2]
    vel_ref[...] = hf[:, 2:5]

    hs = jax.lax.dot_general(smax, ws_ref[...], (((1,), (1,)), ((), ())),
                             preferred_element_type=jnp.float32)
    hs = hs + bs_ref[...]
    sres_ref[...] = hs[:, 0:NUM_SIZE_BINS * 3]
    binl = hs[:, NUM_SIZE_BINS * 3:NUM_SIZE_BINS * 4]
    m = jnp.max(binl, axis=1, keepdims=True)
    e = jnp.exp(binl - m)
    sbin_ref[...] = e / jnp.sum(e, axis=1, keepdims=True)

    cen_ref[...] = sums[:, 0:3] / sums[:, 3:4]


@jax.jit
def kernel(x, raw_xyz, W_vw, b_vw, W_vote, b_vote, W_yaw, b_yaw, W_vel, b_vel,
           W_bin, b_bin, W_sres, b_sres, point2frameidx, frame2batchidx):
    nblk = pl.cdiv(N, R)

    ids = point2frameidx
    starts = jnp.searchsorted(ids, jnp.arange(NF + 1, dtype=jnp.int32)
                              ).astype(jnp.int32)
    bstart = jnp.arange(nblk, dtype=jnp.int32) * R
    blast = jnp.minimum(bstart + R, N) - 1
    blo = ids[bstart]
    bhi = ids[blast]

    wp = jnp.zeros((8, FEAT), jnp.float32)
    wp = wp.at[0:1].set(W_vw).at[1:4].set(W_vote)
    bpt = jnp.zeros((8, 1), jnp.float32)
    bpt = bpt.at[0, 0].set(b_vw[0]).at[1:4, 0].set(b_vote)
    xyzt = jnp.zeros((8, nblk * R), jnp.float32).at[0:3, :N].set(raw_xyz.T)

    grid_spec = pltpu.PrefetchScalarGridSpec(
        num_scalar_prefetch=3,
        grid=(nblk,),
        in_specs=[
            pl.BlockSpec((R, FEAT), lambda b, *_: (b, 0)),
            pl.BlockSpec((8, R), lambda b, *_: (0, b)),
            pl.BlockSpec((8, FEAT), lambda b, *_: (0, 0)),
            pl.BlockSpec((8, 1), lambda b, *_: (0, 0)),
        ],
        out_specs=[
            pl.BlockSpec((NF, FEAT), lambda b, *_: (0, 0)),
            pl.BlockSpec((NF, 4), lambda b, *_: (0, 0)),
        ],
        scratch_shapes=[
            pltpu.VMEM((NCH, FEAT), jnp.float32),
            pltpu.VMEM((NCH, 4), jnp.float32),
            pltpu.VMEM((R, 4), jnp.float32),
            pltpu.VMEM((NF, FEAT), jnp.float32),
            pltpu.VMEM((NF, 4), jnp.float32),
        ],
    )
    fmax, sums = pl.pallas_call(
        _stage1_kernel,
        grid_spec=grid_spec,
        out_shape=[
            jax.ShapeDtypeStruct((NF, FEAT), jnp.float32),
            jax.ShapeDtypeStruct((NF, 4), jnp.float32),
        ],
    )(starts, blo, bhi, x, xyzt, wp, bpt)

    wf = jnp.zeros((8, FEAT), jnp.float32)
    wf = wf.at[0:2].set(W_yaw).at[2:5].set(W_vel)
    bf = jnp.zeros((1, 8), jnp.float32)
    bf = bf.at[0, 0:2].set(b_yaw).at[0, 2:5].set(b_vel)
    ws = jnp.concatenate([W_sres, W_bin], axis=0)
    bs = jnp.concatenate([b_sres, b_bin])[None, :]
    f2b = frame2batchidx[:, None]

    centers, velocities, yaw, sres, sbin = pl.pallas_call(
        _stage2_kernel,
        in_specs=[
            pl.BlockSpec((NF, FEAT), lambda: (0, 0)),
            pl.BlockSpec((NF, 4), lambda: (0, 0)),
            pl.BlockSpec((NF, 1), lambda: (0, 0)),
            pl.BlockSpec((8, FEAT), lambda: (0, 0)),
            pl.BlockSpec((1, 8), lambda: (0, 0)),
            pl.BlockSpec((48, FEAT), lambda: (0, 0)),
            pl.BlockSpec((1, 48), lambda: (0, 0)),
        ],
        out_specs=[
            pl.BlockSpec((NF, 3), lambda: (0, 0)),
            pl.BlockSpec((NF, 3), lambda: (0, 0)),
            pl.BlockSpec((NF, 2), lambda: (0, 0)),
            pl.BlockSpec((NB, NUM_SIZE_BINS * 3), lambda: (0, 0)),
            pl.BlockSpec((NB, NUM_SIZE_BINS), lambda: (0, 0)),
        ],
        out_shape=[
            jax.ShapeDtypeStruct((NF, 3), jnp.float32),
            jax.ShapeDtypeStruct((NF, 3), jnp.float32),
            jax.ShapeDtypeStruct((NF, 2), jnp.float32),
            jax.ShapeDtypeStruct((NB, NUM_SIZE_BINS * 3), jnp.float32),
            jax.ShapeDtypeStruct((NB, NUM_SIZE_BINS), jnp.float32),
        ],
    )(fmax, sums, f2b, wf, bf, ws, bs)

    return (centers, velocities, yaw, sres, sbin)
